# Initial kernel scaffold; baseline (speedup 1.0000x reference)
import jax
import jax.numpy as jnp
from jax.experimental import pallas as pl


def kernel(input, res_input, edge_index, w1, w2, w3, w4, epsilo, b1, b2, b3, b4):
    raise NotImplementedError("write your pallas kernel here")


# speedup vs baseline: 7.8400x; 7.8400x over previous
import jax
import jax.numpy as jnp
from jax import lax
from jax.experimental import pallas as pl
from jax.experimental.pallas import tpu as pltpu
from jax.experimental.pallas import tpu_sc as plsc

N = 10000
E = 320000
D = 128

NC = 2
NS = 16
CHUNK = 80
EDGES_PER_TILE = E // (NC * NS)
NCHUNK = EDGES_PER_TILE // CHUNK

BLK = 2000


def _pre_body(x_ref, r_ref, w1_ref, w2_ref, w3_ref, b2_ref, b3_ref,
              sup_ref, trans_ref, gate1_ref):
    x = x_ref[...]
    sup_ref[...] = jnp.dot(x, w1_ref[...], preferred_element_type=jnp.float32)
    trans_ref[...] = jax.nn.sigmoid(
        jnp.dot(r_ref[...], w2_ref[...], preferred_element_type=jnp.float32)
        + b2_ref[...])
    gate1_ref[...] = (
        jnp.dot(x, w3_ref[...], preferred_element_type=jnp.float32)
        + b3_ref[...])


def _pre(x, r, w1, w2, w3, b2, b3):
    row = pl.BlockSpec((BLK, D), lambda i: (i, 0))
    mat = pl.BlockSpec((D, D), lambda i: (0, 0))
    vec = pl.BlockSpec((1, D), lambda i: (0, 0))
    out = jax.ShapeDtypeStruct((N, D), jnp.float32)
    return pl.pallas_call(
        _pre_body,
        grid=(N // BLK,),
        in_specs=[row, row, mat, mat, mat, vec, vec],
        out_specs=[row, row, row],
        out_shape=[out, out, out],
    )(x, r, w1, w2, w3, b2.reshape(1, D), b3.reshape(1, D))


def _agg_body(sup_hbm, src_hbm, dst_hbm, zeros_hbm, out_hbm,
              src0, dst0, src1, dst1, rows0, rows1, agg_sh, semg0, semg1):
    c = lax.axis_index("c")
    s = lax.axis_index("s")
    base = (c * NS + s) * EDGES_PER_TILE

    def load_idx(k, srcb, dstb):
        off = pl.multiple_of(base + k * CHUNK, 8)
        pltpu.sync_copy(src_hbm.at[pl.ds(off, CHUNK)], srcb)
        pltpu.sync_copy(dst_hbm.at[pl.ds(off, CHUNK)], dstb)

    def issue(srcb, rows, sem):
        pltpu.async_copy(sup_hbm.at[srcb], rows, sem)

    def wait(srcb, rows, sem):
        pltpu.make_async_copy(sup_hbm.at[srcb], rows, sem).wait()

    def flush(rows, dstb):
        pltpu.sync_copy(rows, agg_sh.at[dstb], add=True)

    @pl.when(s == 0)
    def _():
        pltpu.sync_copy(zeros_hbm, agg_sh)
    plsc.subcore_barrier()

    load_idx(0, src0, dst0)
    issue(src0, rows0, semg0)
    load_idx(1, src1, dst1)

    def body(i, carry):
        j = 2 * i
        wait(src0, rows0, semg0)
        issue(src1, rows1, semg1)
        flush(rows0, dst0)
        load_idx(j + 2, src0, dst0)
        wait(src1, rows1, semg1)
        issue(src0, rows0, semg0)
        flush(rows1, dst1)

        @pl.when(j + 3 < NCHUNK)
        def _():
            load_idx(j + 3, src1, dst1)
        return carry

    lax.fori_loop(0, (NCHUNK - 1) // 2, body, 0)
    wait(src0, rows0, semg0)
    flush(rows0, dst0)

    plsc.subcore_barrier()
    rbase = pl.multiple_of(s * 624, 8)

    @pl.when(s < NS - 1)
    def _():
        pltpu.sync_copy(agg_sh.at[pl.ds(rbase, 624)],
                        out_hbm.at[c, pl.ds(rbase, 624)])

    @pl.when(s == NS - 1)
    def _():
        pltpu.sync_copy(agg_sh.at[pl.ds((NS - 1) * 624, 640)],
                        out_hbm.at[c, pl.ds((NS - 1) * 624, 640)])


def _sc_agg(sup, src, dst, zeros):
    mesh = plsc.VectorSubcoreMesh(core_axis_name="c", subcore_axis_name="s")
    f = pl.kernel(
        _agg_body,
        out_type=jax.ShapeDtypeStruct((NC, N, D), jnp.float32),
        mesh=mesh,
        scratch_types=[
            pltpu.VMEM((CHUNK,), jnp.int32),
            pltpu.VMEM((CHUNK,), jnp.int32),
            pltpu.VMEM((CHUNK,), jnp.int32),
            pltpu.VMEM((CHUNK,), jnp.int32),
            pltpu.VMEM((CHUNK, D), jnp.float32),
            pltpu.VMEM((CHUNK, D), jnp.float32),
            pltpu.VMEM_SHARED((N, D), jnp.float32),
            pltpu.SemaphoreType.DMA,
            pltpu.SemaphoreType.DMA,
        ],
    )
    return f(sup, src, dst, zeros)


def _post_body(p0_ref, p1_ref, sup_ref, gate1_ref, trans_ref, w4_ref,
               b1_ref, b4_ref, eps_ref, o1_ref, o2_ref):
    eps = eps_ref[0]
    out = p0_ref[...] + p1_ref[...] + eps * sup_ref[...] + b1_ref[...]
    out = jnp.maximum(out, 0.0)
    gate2 = (jnp.dot(out, w4_ref[...], preferred_element_type=jnp.float32)
             + b4_ref[...])
    gate = jax.nn.sigmoid(gate1_ref[...] + gate2)
    t = trans_ref[...]
    o1_ref[...] = out + gate * (t - out)
    o2_ref[...] = t + gate * (out - t)


def _post(p0, p1, sup, gate1, trans, w4, b1, b4, eps):
    row = pl.BlockSpec((BLK, D), lambda i: (i, 0))
    mat = pl.BlockSpec((D, D), lambda i: (0, 0))
    vec = pl.BlockSpec((1, D), lambda i: (0, 0))
    sca = pl.BlockSpec(memory_space=pltpu.SMEM)
    out = jax.ShapeDtypeStruct((N, D), jnp.float32)
    return pl.pallas_call(
        _post_body,
        grid=(N // BLK,),
        in_specs=[row, row, row, row, row, mat, vec, vec, sca],
        out_specs=[row, row],
        out_shape=[out, out],
    )(p0, p1, sup, gate1, trans, w4, b1.reshape(1, D), b4.reshape(1, D), eps)


def kernel(input, res_input, edge_index, w1, w2, w3, w4, epsilo, b1, b2, b3, b4):
    src = edge_index[0].astype(jnp.int32)
    dst = edge_index[1].astype(jnp.int32)
    zeros = jnp.zeros((N, D), jnp.float32)

    support, trans, gate1 = _pre(input, res_input, w1, w2, w3, b2, b3)
    partials = _sc_agg(support, src, dst, zeros)
    return _post(partials[0], partials[1], support, gate1, trans,
                 w4, b1, b4, epsilo)


# speedup vs baseline: 8.6448x; 1.1027x over previous
import jax
import jax.numpy as jnp
from jax import lax
from jax.experimental import pallas as pl
from jax.experimental.pallas import tpu as pltpu
from jax.experimental.pallas import tpu_sc as plsc

N = 10000
E = 320000
D = 128

NC = 2
NS = 16
CHUNK = 80
EDGES_PER_TILE = E // (NC * NS)
NCHUNK = EDGES_PER_TILE // CHUNK

BLK = 2000


def _pre_body(x_ref, r_ref, w1_ref, w2_ref, w3_ref, b2_ref, b3_ref,
              sup_ref, trans_ref, gate1_ref):
    x = x_ref[...]
    sup_ref[...] = jnp.dot(x, w1_ref[...], preferred_element_type=jnp.float32)
    trans_ref[...] = jax.nn.sigmoid(
        jnp.dot(r_ref[...], w2_ref[...], preferred_element_type=jnp.float32)
        + b2_ref[...])
    gate1_ref[...] = (
        jnp.dot(x, w3_ref[...], preferred_element_type=jnp.float32)
        + b3_ref[...])


def _pre(x, r, w1, w2, w3, b2, b3):
    row = pl.BlockSpec((BLK, D), lambda i: (i, 0))
    mat = pl.BlockSpec((D, D), lambda i: (0, 0))
    vec = pl.BlockSpec((1, D), lambda i: (0, 0))
    out = jax.ShapeDtypeStruct((N, D), jnp.float32)
    return pl.pallas_call(
        _pre_body,
        grid=(N // BLK,),
        in_specs=[row, row, mat, mat, mat, vec, vec],
        out_specs=[row, row, row],
        out_shape=[out, out, out],
    )(x, r, w1, w2, w3, b2.reshape(1, D), b3.reshape(1, D))


def _agg_body(sup_hbm, src_hbm, dst_hbm, zeros_hbm, out_hbm,
              src_idx, dst_idx, rows0, rows1, agg_sh, semg0, semg1):
    c = lax.axis_index("c")
    s = lax.axis_index("s")
    base = pl.multiple_of((c * NS + s) * EDGES_PER_TILE, 8)

    pltpu.sync_copy(src_hbm.at[pl.ds(base, EDGES_PER_TILE)], src_idx)
    pltpu.sync_copy(dst_hbm.at[c, s], dst_idx)

    def issue(j, rows, sem):
        pltpu.async_copy(sup_hbm.at[src_idx.at[pl.ds(j * CHUNK, CHUNK)]],
                         rows, sem)

    def wait(j, rows, sem):
        pltpu.make_async_copy(
            sup_hbm.at[src_idx.at[pl.ds(j * CHUNK, CHUNK)]], rows, sem).wait()

    def flush(j, rows):
        pltpu.sync_copy(rows, agg_sh.at[dst_idx.at[j]], add=True)

    @pl.when(s == 0)
    def _():
        pltpu.sync_copy(zeros_hbm, agg_sh)
    plsc.subcore_barrier()

    issue(0, rows0, semg0)

    def body(i, carry):
        j = 2 * i
        wait(j, rows0, semg0)
        issue(j + 1, rows1, semg1)
        flush(j, rows0)
        wait(j + 1, rows1, semg1)
        issue(j + 2, rows0, semg0)
        flush(j + 1, rows1)
        return carry

    lax.fori_loop(0, (NCHUNK - 1) // 2, body, 0)
    wait(NCHUNK - 1, rows0, semg0)
    flush(NCHUNK - 1, rows0)

    plsc.subcore_barrier()
    rbase = pl.multiple_of(s * 624, 8)

    @pl.when(s < NS - 1)
    def _():
        pltpu.sync_copy(agg_sh.at[pl.ds(rbase, 624)],
                        out_hbm.at[c, pl.ds(rbase, 624)])

    @pl.when(s == NS - 1)
    def _():
        pltpu.sync_copy(agg_sh.at[pl.ds((NS - 1) * 624, 640)],
                        out_hbm.at[c, pl.ds((NS - 1) * 624, 640)])


def _sc_agg(sup, src, dst, zeros):
    mesh = plsc.VectorSubcoreMesh(core_axis_name="c", subcore_axis_name="s")
    f = pl.kernel(
        _agg_body,
        out_type=jax.ShapeDtypeStruct((NC, N, D), jnp.float32),
        mesh=mesh,
        scratch_types=[
            pltpu.VMEM((EDGES_PER_TILE,), jnp.int32),
            pltpu.VMEM((NCHUNK, CHUNK), jnp.int32),
            pltpu.VMEM((CHUNK, D), jnp.float32),
            pltpu.VMEM((CHUNK, D), jnp.float32),
            pltpu.VMEM_SHARED((N, D), jnp.float32),
            pltpu.SemaphoreType.DMA,
            pltpu.SemaphoreType.DMA,
        ],
    )
    return f(sup, src, dst, zeros)


def _post_body(p0_ref, p1_ref, sup_ref, gate1_ref, trans_ref, w4_ref,
               b1_ref, b4_ref, eps_ref, o1_ref, o2_ref):
    eps = eps_ref[0]
    out = p0_ref[...] + p1_ref[...] + eps * sup_ref[...] + b1_ref[...]
    out = jnp.maximum(out, 0.0)
    gate2 = (jnp.dot(out, w4_ref[...], preferred_element_type=jnp.float32)
             + b4_ref[...])
    gate = jax.nn.sigmoid(gate1_ref[...] + gate2)
    t = trans_ref[...]
    o1_ref[...] = out + gate * (t - out)
    o2_ref[...] = t + gate * (out - t)


def _post(p0, p1, sup, gate1, trans, w4, b1, b4, eps):
    row = pl.BlockSpec((BLK, D), lambda i: (i, 0))
    mat = pl.BlockSpec((D, D), lambda i: (0, 0))
    vec = pl.BlockSpec((1, D), lambda i: (0, 0))
    sca = pl.BlockSpec(memory_space=pltpu.SMEM)
    out = jax.ShapeDtypeStruct((N, D), jnp.float32)
    return pl.pallas_call(
        _post_body,
        grid=(N // BLK,),
        in_specs=[row, row, row, row, row, mat, vec, vec, sca],
        out_specs=[row, row],
        out_shape=[out, out],
    )(p0, p1, sup, gate1, trans, w4, b1.reshape(1, D), b4.reshape(1, D), eps)


def kernel(input, res_input, edge_index, w1, w2, w3, w4, epsilo, b1, b2, b3, b4):
    src = edge_index[0].astype(jnp.int32)
    dst = edge_index[1].astype(jnp.int32).reshape(NC, NS, NCHUNK, CHUNK)
    zeros = jnp.zeros((N, D), jnp.float32)

    support, trans, gate1 = _pre(input, res_input, w1, w2, w3, b2, b3)
    partials = _sc_agg(support, src, dst, zeros)
    return _post(partials[0], partials[1], support, gate1, trans,
                 w4, b1, b4, epsilo)
